# baseline (speedup 1.0000x reference)
import jax
import jax.numpy as jnp
from jax.experimental import pallas as pl
from jax.experimental.pallas import tpu as pltpu

_B = 16384
_W = 2048
_NNZ = 128
_NCYC = 32
_BLK = 256
_GRID = _B // _BLK


def _body(signs_ref, g_ref, r_ref, c_ref, s_ref,
          c_out_ref, s_out_ref, v_ref, acc_ref):
    i = pl.program_id(0)

    cb = c_ref[...]
    sb = s_ref[...]
    c_out_ref[...] = cb
    s_out_ref[...] = sb

    g = g_ref[...]
    cg = jax.lax.dot_general(cb, g, (((1,), (0,)), ((), ())),
                             preferred_element_type=jnp.float32)
    sg = jax.lax.dot_general(sb, g, (((1,), (0,)), ((), ())),
                             preferred_element_type=jnp.float32)
    sg = sg * signs_ref[...]

    ang = jnp.arctan2(sg, cg)

    pc = jax.lax.dot_general(ang, r_ref[...], (((1,), (0,)), ((), ())),
                             preferred_element_type=jnp.float32)
    part = jnp.sum(jnp.abs(pc))

    @pl.when(i == 0)
    def _():
        acc_ref[0, 0] = 0.0

    acc_ref[0, 0] += part

    @pl.when(i == _GRID - 1)
    def _():
        v_ref[0, 0] = acc_ref[0, 0] * (1.0 / (_B * _NCYC))


def kernel(c, s, cyinds, cysigns, cyrows):
    signs = cysigns.reshape(1, _NNZ)
    gather_oh = (jax.lax.broadcasted_iota(jnp.int32, (_W, _NNZ), 0)
                 == cyinds[None, :]).astype(jnp.float32)
    seg_oh = (jax.lax.broadcasted_iota(jnp.int32, (_NNZ, _NCYC), 1)
              == cyrows[:, None]).astype(jnp.float32)

    c_out, s_out, v = pl.pallas_call(
        _body,
        grid=(_GRID,),
        in_specs=[
            pl.BlockSpec((1, _NNZ), lambda i: (0, 0)),
            pl.BlockSpec((_W, _NNZ), lambda i: (0, 0)),
            pl.BlockSpec((_NNZ, _NCYC), lambda i: (0, 0)),
            pl.BlockSpec((_BLK, _W), lambda i: (i, 0)),
            pl.BlockSpec((_BLK, _W), lambda i: (i, 0)),
        ],
        out_specs=[
            pl.BlockSpec((_BLK, _W), lambda i: (i, 0)),
            pl.BlockSpec((_BLK, _W), lambda i: (i, 0)),
            pl.BlockSpec((1, 1), lambda i: (0, 0),
                         memory_space=pltpu.SMEM),
        ],
        out_shape=[
            jax.ShapeDtypeStruct((_B, _W), jnp.float32),
            jax.ShapeDtypeStruct((_B, _W), jnp.float32),
            jax.ShapeDtypeStruct((1, 1), jnp.float32),
        ],
        scratch_shapes=[pltpu.SMEM((1, 1), jnp.float32)],
    )(signs, gather_oh, seg_oh, c, s)

    return (c_out, s_out, v[0, 0])


# speedup vs baseline: 1.0595x; 1.0595x over previous
import jax
import jax.numpy as jnp
from jax.experimental import pallas as pl
from jax.experimental.pallas import tpu as pltpu

_B = 16384
_W = 2048
_NNZ = 128
_NCYC = 32
_BLK = 512
_GRID = _B // _BLK


def _body(signs_ref, inds_ref, rows_ref, c_ref, s_ref,
          c_out_ref, s_out_ref, v_ref, g_ref, rt_ref, acc_ref):
    i = pl.program_id(0)

    @pl.when(i == 0)
    def _():
        acc_ref[0, 0] = 0.0
        g_ref[...] = (jax.lax.broadcasted_iota(jnp.int32, (_W, _NNZ), 0)
                      == inds_ref[...]).astype(jnp.float32)
        rt_ref[...] = (jax.lax.broadcasted_iota(jnp.int32, (_NCYC, _NNZ), 0)
                       == rows_ref[...]).astype(jnp.float32)

    cb = c_ref[...]
    sb = s_ref[...]
    c_out_ref[...] = cb
    s_out_ref[...] = sb

    g = g_ref[...]
    cg = jax.lax.dot_general(cb, g, (((1,), (0,)), ((), ())),
                             preferred_element_type=jnp.float32)
    sg = jax.lax.dot_general(sb, g, (((1,), (0,)), ((), ())),
                             preferred_element_type=jnp.float32)
    sg = sg * signs_ref[...]

    ang = jnp.arctan2(sg, cg)

    pc = jax.lax.dot_general(ang, rt_ref[...], (((1,), (1,)), ((), ())),
                             preferred_element_type=jnp.float32)
    part = jnp.sum(jnp.abs(pc))

    acc_ref[0, 0] += part

    @pl.when(i == _GRID - 1)
    def _():
        v_ref[0, 0] = acc_ref[0, 0] * (1.0 / (_B * _NCYC))


def kernel(c, s, cyinds, cysigns, cyrows):
    signs = cysigns.reshape(1, _NNZ)
    inds = cyinds.reshape(1, _NNZ)
    rows = cyrows.reshape(1, _NNZ)

    c_out, s_out, v = pl.pallas_call(
        _body,
        grid=(_GRID,),
        in_specs=[
            pl.BlockSpec((1, _NNZ), lambda i: (0, 0)),
            pl.BlockSpec((1, _NNZ), lambda i: (0, 0)),
            pl.BlockSpec((1, _NNZ), lambda i: (0, 0)),
            pl.BlockSpec((_BLK, _W), lambda i: (i, 0)),
            pl.BlockSpec((_BLK, _W), lambda i: (i, 0)),
        ],
        out_specs=[
            pl.BlockSpec((_BLK, _W), lambda i: (i, 0)),
            pl.BlockSpec((_BLK, _W), lambda i: (i, 0)),
            pl.BlockSpec((1, 1), lambda i: (0, 0),
                         memory_space=pltpu.SMEM),
        ],
        out_shape=[
            jax.ShapeDtypeStruct((_B, _W), jnp.float32),
            jax.ShapeDtypeStruct((_B, _W), jnp.float32),
            jax.ShapeDtypeStruct((1, 1), jnp.float32),
        ],
        scratch_shapes=[
            pltpu.VMEM((_W, _NNZ), jnp.float32),
            pltpu.VMEM((_NCYC, _NNZ), jnp.float32),
            pltpu.SMEM((1, 1), jnp.float32),
        ],
    )(signs, inds, rows, c, s)

    return (c_out, s_out, v[0, 0])
